# baseline (device time: 232859 ns/iter reference)
import jax
import jax.numpy as jnp
from jax import lax
from jax.experimental import pallas as pl
from jax.experimental.pallas import tpu as pltpu

N_X = 2
CX = 16
N_SLOTS = 12


def kernel(x):
    _, m, n2 = x.shape
    n = n2 // N_X
    half = m // 2
    rows = half // CX

    def body(x_hbm, out_ref, xcomm, ycomm, local_sem,
             xsend_sems, xrecv_sems, ysend_sems, yrecv_sems,
             xcredit_sem, ycredit_sem):
        my_x = lax.axis_index("x")
        my_y = lax.axis_index("y")
        my_z = lax.axis_index("z")
        partner = (1 - my_x, my_y, my_z)
        buddy = (my_x, 1 - my_y, my_z)

        barrier_sem = pltpu.get_barrier_semaphore()
        for peer in (partner, buddy):
            pl.semaphore_signal(
                barrier_sem, inc=1,
                device_id=peer,
                device_id_type=pl.DeviceIdType.MESH,
            )
        pl.semaphore_wait(barrier_sem, 2)

        local_copy = pltpu.make_async_copy(
            x_hbm.at[0, :, pl.ds(my_x * n, n)],
            out_ref,
            local_sem,
        )
        local_copy.start()

        def make_x_rdma(k):
            return pltpu.make_async_remote_copy(
                src_ref=x_hbm.at[0, pl.ds(my_y * half + k * rows, rows),
                                 pl.ds((1 - my_x) * n, n)],
                dst_ref=xcomm.at[k % N_SLOTS],
                send_sem=xsend_sems.at[k],
                recv_sem=xrecv_sems.at[k % N_SLOTS],
                device_id=partner,
                device_id_type=pl.DeviceIdType.MESH,
            )

        def make_y_rdma(k):
            return pltpu.make_async_remote_copy(
                src_ref=xcomm.at[k % N_SLOTS],
                dst_ref=ycomm.at[k % N_SLOTS],
                send_sem=ysend_sems.at[k],
                recv_sem=yrecv_sems.at[k % N_SLOTS],
                device_id=buddy,
                device_id_type=pl.DeviceIdType.MESH,
            )

        x_rdmas = [make_x_rdma(k) for k in range(CX)]
        y_rdmas = [make_y_rdma(k) for k in range(CX)]

        for k in range(N_SLOTS):
            x_rdmas[k].start()

        local_copy.wait()

        LAG = 2

        def consume_y(j):
            y_rdmas[j].wait_send()
            if j + N_SLOTS < CX:
                pl.semaphore_signal(
                    xcredit_sem, inc=1,
                    device_id=partner,
                    device_id_type=pl.DeviceIdType.MESH,
                )
                pl.semaphore_wait(xcredit_sem, 1)
                x_rdmas[j + N_SLOTS].start()
            y_rdmas[j].wait_recv()
            yr = pl.ds((1 - my_y) * half + j * rows, rows)
            out_ref[yr, :] = out_ref[yr, :] + ycomm[j % N_SLOTS, :, :]
            if j + N_SLOTS < CX:
                pl.semaphore_signal(
                    ycredit_sem, inc=1,
                    device_id=buddy,
                    device_id_type=pl.DeviceIdType.MESH,
                )

        for k in range(CX):
            x_rdmas[k].wait_recv()
            if k >= N_SLOTS:
                pl.semaphore_wait(ycredit_sem, 1)
            y_rdmas[k].start()
            xr = pl.ds(my_y * half + k * rows, rows)
            out_ref[xr, :] = out_ref[xr, :] + xcomm[k % N_SLOTS, :, :]
            if k >= LAG:
                consume_y(k - LAG)

        for j in range(CX - LAG, CX):
            consume_y(j)
        for k in range(CX):
            x_rdmas[k].wait_send()

    return pl.pallas_call(
        body,
        out_shape=jax.ShapeDtypeStruct((m, n), jnp.float32),
        in_specs=[pl.BlockSpec(memory_space=pl.ANY)],
        out_specs=pl.BlockSpec(memory_space=pltpu.VMEM),
        scratch_shapes=[
            pltpu.VMEM((N_SLOTS, rows, n), jnp.float32),
            pltpu.VMEM((N_SLOTS, rows, n), jnp.float32),
            pltpu.SemaphoreType.DMA,
            pltpu.SemaphoreType.DMA((CX,)),
            pltpu.SemaphoreType.DMA((N_SLOTS,)),
            pltpu.SemaphoreType.DMA((CX,)),
            pltpu.SemaphoreType.DMA((N_SLOTS,)),
            pltpu.SemaphoreType.REGULAR,
            pltpu.SemaphoreType.REGULAR,
        ],
        compiler_params=pltpu.CompilerParams(
            collective_id=0,
            vmem_limit_bytes=61 * 1024 * 1024,
        ),
    )(x)


# device time: 222973 ns/iter; 1.0443x vs baseline; 1.0443x over previous
import jax
import jax.numpy as jnp
from jax import lax
from jax.experimental import pallas as pl
from jax.experimental.pallas import tpu as pltpu

N_X = 2
CX = 16
N_SLOTS = 12


def kernel(x):
    _, m, n2 = x.shape
    n = n2 // N_X
    half = m // 2
    rows = half // CX

    def body(x_hbm, out_hbm, acc, xcomm, ycomm, local_sem,
             xsend_sems, xrecv_sems, ysend_sems, yrecv_sems,
             xout_sems, yout_sems, xcredit_sem, ycredit_sem):
        my_x = lax.axis_index("x")
        my_y = lax.axis_index("y")
        my_z = lax.axis_index("z")
        partner = (1 - my_x, my_y, my_z)
        buddy = (my_x, 1 - my_y, my_z)

        barrier_sem = pltpu.get_barrier_semaphore()
        for peer in (partner, buddy):
            pl.semaphore_signal(
                barrier_sem, inc=1,
                device_id=peer,
                device_id_type=pl.DeviceIdType.MESH,
            )
        pl.semaphore_wait(barrier_sem, 2)

        local_copy = pltpu.make_async_copy(
            x_hbm.at[0, :, pl.ds(my_x * n, n)],
            acc,
            local_sem,
        )
        local_copy.start()

        def make_x_rdma(k):
            return pltpu.make_async_remote_copy(
                src_ref=x_hbm.at[0, pl.ds(my_y * half + k * rows, rows),
                                 pl.ds((1 - my_x) * n, n)],
                dst_ref=xcomm.at[k % N_SLOTS],
                send_sem=xsend_sems.at[k],
                recv_sem=xrecv_sems.at[k % N_SLOTS],
                device_id=partner,
                device_id_type=pl.DeviceIdType.MESH,
            )

        def make_y_rdma(k):
            return pltpu.make_async_remote_copy(
                src_ref=xcomm.at[k % N_SLOTS],
                dst_ref=ycomm.at[k % N_SLOTS],
                send_sem=ysend_sems.at[k],
                recv_sem=yrecv_sems.at[k % N_SLOTS],
                device_id=buddy,
                device_id_type=pl.DeviceIdType.MESH,
            )

        x_rdmas = [make_x_rdma(k) for k in range(CX)]
        y_rdmas = [make_y_rdma(k) for k in range(CX)]
        out_copies = []

        for k in range(N_SLOTS):
            x_rdmas[k].start()

        local_copy.wait()

        def flush_out(row_start, sem):
            cp = pltpu.make_async_copy(
                acc.at[pl.ds(row_start, rows), :],
                out_hbm.at[pl.ds(row_start, rows), :],
                sem,
            )
            cp.start()
            out_copies.append(cp)

        LAG = 2

        def consume_y(j):
            y_rdmas[j].wait_send()
            if j + N_SLOTS < CX:
                pl.semaphore_signal(
                    xcredit_sem, inc=1,
                    device_id=partner,
                    device_id_type=pl.DeviceIdType.MESH,
                )
                pl.semaphore_wait(xcredit_sem, 1)
                x_rdmas[j + N_SLOTS].start()
            y_rdmas[j].wait_recv()
            yr0 = (1 - my_y) * half + j * rows
            yr = pl.ds(yr0, rows)
            acc[yr, :] = acc[yr, :] + ycomm[j % N_SLOTS, :, :]
            flush_out(yr0, yout_sems.at[j])
            if j + N_SLOTS < CX:
                pl.semaphore_signal(
                    ycredit_sem, inc=1,
                    device_id=buddy,
                    device_id_type=pl.DeviceIdType.MESH,
                )

        for k in range(CX):
            x_rdmas[k].wait_recv()
            if k >= N_SLOTS:
                pl.semaphore_wait(ycredit_sem, 1)
            y_rdmas[k].start()
            xr0 = my_y * half + k * rows
            xr = pl.ds(xr0, rows)
            acc[xr, :] = acc[xr, :] + xcomm[k % N_SLOTS, :, :]
            flush_out(xr0, xout_sems.at[k])
            if k >= LAG:
                consume_y(k - LAG)

        for j in range(CX - LAG, CX):
            consume_y(j)
        for k in range(CX):
            x_rdmas[k].wait_send()
        for cp in out_copies:
            cp.wait()

    return pl.pallas_call(
        body,
        out_shape=jax.ShapeDtypeStruct((m, n), jnp.float32),
        in_specs=[pl.BlockSpec(memory_space=pl.ANY)],
        out_specs=pl.BlockSpec(memory_space=pl.ANY),
        scratch_shapes=[
            pltpu.VMEM((m, n), jnp.float32),
            pltpu.VMEM((N_SLOTS, rows, n), jnp.float32),
            pltpu.VMEM((N_SLOTS, rows, n), jnp.float32),
            pltpu.SemaphoreType.DMA,
            pltpu.SemaphoreType.DMA((CX,)),
            pltpu.SemaphoreType.DMA((N_SLOTS,)),
            pltpu.SemaphoreType.DMA((CX,)),
            pltpu.SemaphoreType.DMA((N_SLOTS,)),
            pltpu.SemaphoreType.DMA((CX,)),
            pltpu.SemaphoreType.DMA((CX,)),
            pltpu.SemaphoreType.REGULAR,
            pltpu.SemaphoreType.REGULAR,
        ],
        compiler_params=pltpu.CompilerParams(
            collective_id=0,
            vmem_limit_bytes=61 * 1024 * 1024,
        ),
    )(x)
